# baseline (speedup 1.0000x reference)
import jax
import jax.numpy as jnp
from jax import lax
from jax.experimental import pallas as pl
from jax.experimental.pallas import tpu as pltpu

_ALPHA = 0.25
_REG_MAX = 16
_EPS = 1e-07

_ROWS = 32768
_BR = 512
_GRID = _ROWS // _BR
_TR = _BR // 16


def _body(pred_ref, t_ref, out_ref):
    i = pl.program_id(0)

    t8 = t_ref[...]
    p8 = jnp.where(
        lax.broadcasted_iota(jnp.int32, (8, 128), 0)
        == (lax.broadcasted_iota(jnp.int32, (8, 128), 1) // _REG_MAX),
        1.0, 0.0).astype(jnp.float32)
    t_e = jax.lax.dot(t8, p8, preferred_element_type=jnp.float32)

    coord_e = t_e * jnp.float32(_REG_MAX - 1)
    valid = (coord_e >= 0.0) & (coord_e < jnp.float32(_REG_MAX - 1))

    kf = (lax.broadcasted_iota(jnp.int32, (_BR, 128), 1)
          & (_REG_MAX - 1)).astype(jnp.float32)

    p = pred_ref[...]
    omp = 1.0 - p
    lg = jnp.log(p + jnp.float32(_EPS))
    w = jnp.maximum(1.0 - jnp.abs(coord_e - kf), 0.0)
    w = jnp.where(valid, w, 0.0)
    contrib = (w * (omp * omp)) * lg

    psum = jnp.sum(contrib.reshape(_BR // 8, 8, 128), axis=0)

    @pl.when(i == 0)
    def _():
        out_ref[...] = psum

    @pl.when(i != 0)
    def _():
        out_ref[...] += psum


def kernel(pred_dist, target_boxes):
    pred32 = pred_dist.reshape(_ROWS, 128)
    t8 = target_boxes.reshape(_ROWS, 8)
    out = pl.pallas_call(
        _body,
        grid=(_GRID,),
        in_specs=[
            pl.BlockSpec((_BR, 128), lambda i: (i, 0)),
            pl.BlockSpec((_BR, 8), lambda i: (i, 0)),
        ],
        out_specs=pl.BlockSpec((8, 128), lambda i: (0, 0)),
        out_shape=jax.ShapeDtypeStruct((8, 128), jnp.float32),
        compiler_params=pltpu.CompilerParams(
            dimension_semantics=("arbitrary",)),
    )(pred32, t8)
    return jnp.sum(out) * jnp.float32(-_ALPHA / (_ROWS * 128))


# speedup vs baseline: 8.7562x; 8.7562x over previous
import jax
import jax.numpy as jnp
from jax import lax
from jax.experimental import pallas as pl
from jax.experimental.pallas import tpu as pltpu

_ALPHA = 0.25
_REG_MAX = 16
_EPS = 1e-07

_BOXES = 65536
_WB = 2048
_GRID = _BOXES // _WB
_W = _WB // 128


def _body(pred_ref, t_ref, out_ref):
    i = pl.program_id(0)

    kf = lax.broadcasted_iota(
        jnp.int32, (_REG_MAX, 128), 0).astype(jnp.float32)
    t3 = t_ref[...].reshape(_W, 4, 128)

    acc = jnp.zeros((_REG_MAX, 128), jnp.float32)
    for c in range(4):
        for j in range(_W):
            coord = t3[j, c][None, :] * jnp.float32(_REG_MAX - 1)
            coord = jnp.where(
                (coord >= 0.0) & (coord < jnp.float32(_REG_MAX - 1)),
                coord, 1e9)
            cb = jnp.broadcast_to(coord, (_REG_MAX, 128))
            p = pred_ref[c, :, 128 * j:128 * (j + 1)]
            omp = 1.0 - p
            lg = jnp.log(p + jnp.float32(_EPS))
            w = jnp.maximum(1.0 - jnp.abs(cb - kf), 0.0)
            acc = acc + (w * (omp * omp)) * lg

    @pl.when(i == 0)
    def _():
        out_ref[...] = acc

    @pl.when(i != 0)
    def _():
        out_ref[...] += acc


def kernel(pred_dist, target_boxes):
    pred_t = jnp.transpose(pred_dist, (1, 2, 0))
    t_v = (target_boxes.reshape(_BOXES // 128, 128, 4)
           .transpose(0, 2, 1).reshape(_BOXES * 4 // 128, 128))
    out = pl.pallas_call(
        _body,
        grid=(_GRID,),
        in_specs=[
            pl.BlockSpec((4, _REG_MAX, _WB), lambda i: (0, 0, i)),
            pl.BlockSpec((4 * _W, 128), lambda i: (i, 0)),
        ],
        out_specs=pl.BlockSpec((_REG_MAX, 128), lambda i: (0, 0)),
        out_shape=jax.ShapeDtypeStruct((_REG_MAX, 128), jnp.float32),
        compiler_params=pltpu.CompilerParams(
            dimension_semantics=("arbitrary",)),
    )(pred_t, t_v)
    return jnp.sum(out) * jnp.float32(-_ALPHA / (_BOXES * 4 * _REG_MAX))


# speedup vs baseline: 8.7604x; 1.0005x over previous
import jax
import jax.numpy as jnp
from jax import lax
from jax.experimental import pallas as pl
from jax.experimental.pallas import tpu as pltpu

_ALPHA = 0.25
_REG_MAX = 16
_EPS = 1e-07

_BOXES = 65536
_WB = 2048
_GRID = _BOXES // _WB
_W = _WB // 128


def _body(pred_ref, t_ref, out_ref):
    i = pl.program_id(0)

    kf = lax.broadcasted_iota(
        jnp.int32, (_REG_MAX, 128), 0).astype(jnp.float32)
    t3 = t_ref[...].reshape(_W, 4, 128)

    acc = jnp.zeros((_REG_MAX, 128), jnp.float32)
    for c in range(4):
        for j in range(_W):
            coord = t3[j, c][None, :] * jnp.float32(_REG_MAX - 1)
            coord = jnp.where(
                (coord >= 0.0) & (coord < jnp.float32(_REG_MAX - 1)),
                coord, 1e9)
            cb = jnp.broadcast_to(coord, (_REG_MAX, 128))
            p = pred_ref[c, :, 128 * j:128 * (j + 1)]
            omp = 1.0 - p
            lg = jnp.log(p + jnp.float32(_EPS))
            w = jnp.maximum(1.0 - jnp.abs(cb - kf), 0.0)
            acc = acc + (w * (omp * omp)) * lg

    @pl.when(i == 0)
    def _():
        out_ref[...] = acc

    @pl.when(i != 0)
    def _():
        out_ref[...] += acc


def kernel(pred_dist, target_boxes):
    pred_t = jnp.transpose(pred_dist, (1, 2, 0))
    t_v = (target_boxes.reshape(_BOXES // 128, 128, 4)
           .transpose(0, 2, 1).reshape(_BOXES * 4 // 128, 128))
    pred_t = pltpu.with_memory_space_constraint(pred_t, pltpu.MemorySpace.HBM)
    t_v = pltpu.with_memory_space_constraint(t_v, pltpu.MemorySpace.HBM)
    out = pl.pallas_call(
        _body,
        grid=(_GRID,),
        in_specs=[
            pl.BlockSpec((4, _REG_MAX, _WB), lambda i: (0, 0, i)),
            pl.BlockSpec((4 * _W, 128), lambda i: (i, 0)),
        ],
        out_specs=pl.BlockSpec((_REG_MAX, 128), lambda i: (0, 0)),
        out_shape=jax.ShapeDtypeStruct((_REG_MAX, 128), jnp.float32),
        compiler_params=pltpu.CompilerParams(
            dimension_semantics=("arbitrary",)),
    )(pred_t, t_v)
    return jnp.sum(out) * jnp.float32(-_ALPHA / (_BOXES * 4 * _REG_MAX))


# speedup vs baseline: 12.4083x; 1.4164x over previous
import jax
import jax.numpy as jnp
from jax import lax
from jax.experimental import pallas as pl
from jax.experimental.pallas import tpu as pltpu

_ALPHA = 0.25
_REG_MAX = 16
_EPS = 1e-07

_BOXES = 65536
_WB = 4096
_GRID = _BOXES // _WB
_W = _WB // 128


def _body(pred_ref, t_ref, out_ref):
    i = pl.program_id(0)

    kf = lax.broadcasted_iota(
        jnp.int32, (_REG_MAX, 128), 0).astype(jnp.float32)
    t3 = t_ref[...].reshape(_W, 4, 128)

    acc = jnp.zeros((_REG_MAX, 128), jnp.float32)
    for c in range(4):
        for j in range(_W):
            coord = t3[j, c][None, :] * jnp.float32(_REG_MAX - 1)
            coord = jnp.where(
                (coord >= 0.0) & (coord < jnp.float32(_REG_MAX - 1)),
                coord, 1e9)
            cb = jnp.broadcast_to(coord, (_REG_MAX, 128))
            p = pred_ref[c, :, 128 * j:128 * (j + 1)]
            omp = 1.0 - p
            lg = jnp.log(p + jnp.float32(_EPS))
            w = jnp.maximum(1.0 - jnp.abs(cb - kf), 0.0)
            acc = acc + (w * (omp * omp)) * lg

    @pl.when(i == 0)
    def _():
        out_ref[...] = acc

    @pl.when(i != 0)
    def _():
        out_ref[...] += acc


def kernel(pred_dist, target_boxes):
    pred_t = jnp.transpose(pred_dist, (1, 2, 0))
    t_v = (target_boxes.reshape(_BOXES // 128, 128, 4)
           .transpose(0, 2, 1).reshape(_BOXES * 4 // 128, 128))
    pred_t = pltpu.with_memory_space_constraint(pred_t, pltpu.MemorySpace.HBM)
    t_v = pltpu.with_memory_space_constraint(t_v, pltpu.MemorySpace.HBM)
    out = pl.pallas_call(
        _body,
        grid=(_GRID,),
        in_specs=[
            pl.BlockSpec((4, _REG_MAX, _WB), lambda i: (0, 0, i)),
            pl.BlockSpec((4 * _W, 128), lambda i: (i, 0)),
        ],
        out_specs=pl.BlockSpec((_REG_MAX, 128), lambda i: (0, 0)),
        out_shape=jax.ShapeDtypeStruct((_REG_MAX, 128), jnp.float32),
        compiler_params=pltpu.CompilerParams(
            dimension_semantics=("arbitrary",)),
    )(pred_t, t_v)
    return jnp.sum(out) * jnp.float32(-_ALPHA / (_BOXES * 4 * _REG_MAX))


# speedup vs baseline: 15.5243x; 1.2511x over previous
import jax
import jax.numpy as jnp
from jax import lax
from jax.experimental import pallas as pl
from jax.experimental.pallas import tpu as pltpu

_ALPHA = 0.25
_REG_MAX = 16
_EPS = 1e-07

_BOXES = 65536
_WB = 8192
_GRID = _BOXES // _WB
_W = _WB // 128


def _body(pred_ref, t_ref, out_ref):
    i = pl.program_id(0)

    kf = lax.broadcasted_iota(
        jnp.int32, (_REG_MAX, 128), 0).astype(jnp.float32)
    t3 = t_ref[...].reshape(_W, 4, 128)

    acc = jnp.zeros((_REG_MAX, 128), jnp.float32)
    for c in range(4):
        for j in range(_W):
            coord = t3[j, c][None, :] * jnp.float32(_REG_MAX - 1)
            coord = jnp.where(
                (coord >= 0.0) & (coord < jnp.float32(_REG_MAX - 1)),
                coord, 1e9)
            cb = jnp.broadcast_to(coord, (_REG_MAX, 128))
            p = pred_ref[c, :, 128 * j:128 * (j + 1)]
            omp = 1.0 - p
            lg = jnp.log(p + jnp.float32(_EPS))
            w = jnp.maximum(1.0 - jnp.abs(cb - kf), 0.0)
            acc = acc + (w * (omp * omp)) * lg

    @pl.when(i == 0)
    def _():
        out_ref[...] = acc

    @pl.when(i != 0)
    def _():
        out_ref[...] += acc


def kernel(pred_dist, target_boxes):
    pred_t = jnp.transpose(pred_dist, (1, 2, 0))
    t_v = (target_boxes.reshape(_BOXES // 128, 128, 4)
           .transpose(0, 2, 1).reshape(_BOXES * 4 // 128, 128))
    pred_t = pltpu.with_memory_space_constraint(pred_t, pltpu.MemorySpace.HBM)
    t_v = pltpu.with_memory_space_constraint(t_v, pltpu.MemorySpace.HBM)
    out = pl.pallas_call(
        _body,
        grid=(_GRID,),
        in_specs=[
            pl.BlockSpec((4, _REG_MAX, _WB), lambda i: (0, 0, i)),
            pl.BlockSpec((4 * _W, 128), lambda i: (i, 0)),
        ],
        out_specs=pl.BlockSpec((_REG_MAX, 128), lambda i: (0, 0)),
        out_shape=jax.ShapeDtypeStruct((_REG_MAX, 128), jnp.float32),
        compiler_params=pltpu.CompilerParams(
            dimension_semantics=("arbitrary",)),
    )(pred_t, t_v)
    return jnp.sum(out) * jnp.float32(-_ALPHA / (_BOXES * 4 * _REG_MAX))


# speedup vs baseline: 16.9082x; 1.0891x over previous
import jax
import jax.numpy as jnp
from jax import lax
from jax.experimental import pallas as pl
from jax.experimental.pallas import tpu as pltpu

_ALPHA = 0.25
_REG_MAX = 16
_EPS = 1e-07

_BOXES = 65536
_WB = 16384
_GRID = _BOXES // _WB
_W = _WB // 128


def _body(pred_ref, t_ref, out_ref):
    i = pl.program_id(0)

    kf = lax.broadcasted_iota(
        jnp.int32, (_REG_MAX, 128), 0).astype(jnp.float32)
    t3 = t_ref[...].reshape(_W, 4, 128)

    acc = jnp.zeros((_REG_MAX, 128), jnp.float32)
    for c in range(4):
        for j in range(_W):
            coord = t3[j, c][None, :] * jnp.float32(_REG_MAX - 1)
            coord = jnp.where(
                (coord >= 0.0) & (coord < jnp.float32(_REG_MAX - 1)),
                coord, 1e9)
            cb = jnp.broadcast_to(coord, (_REG_MAX, 128))
            p = pred_ref[c, :, 128 * j:128 * (j + 1)]
            omp = 1.0 - p
            lg = jnp.log(p + jnp.float32(_EPS))
            w = jnp.maximum(1.0 - jnp.abs(cb - kf), 0.0)
            acc = acc + (w * (omp * omp)) * lg

    @pl.when(i == 0)
    def _():
        out_ref[...] = acc

    @pl.when(i != 0)
    def _():
        out_ref[...] += acc


def kernel(pred_dist, target_boxes):
    pred_t = jnp.transpose(pred_dist, (1, 2, 0))
    t_v = (target_boxes.reshape(_BOXES // 128, 128, 4)
           .transpose(0, 2, 1).reshape(_BOXES * 4 // 128, 128))
    pred_t = pltpu.with_memory_space_constraint(pred_t, pltpu.MemorySpace.HBM)
    t_v = pltpu.with_memory_space_constraint(t_v, pltpu.MemorySpace.HBM)
    out = pl.pallas_call(
        _body,
        grid=(_GRID,),
        in_specs=[
            pl.BlockSpec((4, _REG_MAX, _WB), lambda i: (0, 0, i)),
            pl.BlockSpec((4 * _W, 128), lambda i: (i, 0)),
        ],
        out_specs=pl.BlockSpec((_REG_MAX, 128), lambda i: (0, 0)),
        out_shape=jax.ShapeDtypeStruct((_REG_MAX, 128), jnp.float32),
        compiler_params=pltpu.CompilerParams(
            dimension_semantics=("arbitrary",)),
    )(pred_t, t_v)
    return jnp.sum(out) * jnp.float32(-_ALPHA / (_BOXES * 4 * _REG_MAX))
